# baseline (speedup 1.0000x reference)
import functools
import math

import jax
import jax.numpy as jnp
from jax import lax
from jax.experimental import pallas as pl
from jax.experimental.pallas import tpu as pltpu
from jax.experimental.pallas import tpu_sc as plsc

_LS = 0.1
_CONF = 1.0 - _LS
_RBLK = 4096
_A = 4096
_B = 49152
_CHR = 32

_NC = 2
_NS = 16
_L = 16
_NW = _NC * _NS


def _sc_body(b, n, xT, tgt, acc_out, tacc_out, tgt_v, buf, acc_v, tacc_v,
             sem0, sem1):
    w_per = (_B - _A) // _NW
    nchunks = w_per // _CHR
    ngroups = b // _L
    wid = lax.axis_index("s") * _NC + lax.axis_index("c")
    c_base = _A + wid * w_per

    pltpu.sync_copy(tgt, tgt_v)

    @pl.loop(0, ngroups)
    def _zero(j):
        off = j * _L
        acc_v[pl.ds(off, _L)] = jnp.zeros((_L,), jnp.float32)
        tacc_v[pl.ds(off, _L)] = jnp.zeros((_L,), jnp.float32)

    def start(k, bsel, sem):
        pltpu.make_async_copy(
            xT.at[pl.ds(c_base + k * _CHR, _CHR), :],
            buf.at[bsel], sem).start()

    def wait(bsel, sem):
        pltpu.make_async_copy(
            xT.at[pl.ds(0, _CHR), :], buf.at[bsel], sem).wait()

    def compute(bufref, c0):
        @pl.loop(0, ngroups)
        def _grp(j):
            off = j * _L
            a0 = bufref[0, pl.ds(off, _L)]
            a1 = bufref[1, pl.ds(off, _L)]
            a2 = bufref[2, pl.ds(off, _L)]
            a3 = bufref[3, pl.ds(off, _L)]
            for r in range(4, _CHR, 4):
                a0 = a0 + bufref[r, pl.ds(off, _L)]
                a1 = a1 + bufref[r + 1, pl.ds(off, _L)]
                a2 = a2 + bufref[r + 2, pl.ds(off, _L)]
                a3 = a3 + bufref[r + 3, pl.ds(off, _L)]
            acc_v[pl.ds(off, _L)] = (acc_v[pl.ds(off, _L)]
                                     + (a0 + a1) + (a2 + a3))

            t16 = tgt_v[pl.ds(off, _L)]
            rel = t16 - c0
            valid = (rel >= 0) & (rel < _CHR)
            relc = jnp.minimum(jnp.maximum(rel, 0), _CHR - 1)
            bcol = off + lax.iota(jnp.int32, _L)
            xt = plsc.load_gather(bufref, [relc, bcol])
            zero = jnp.zeros((_L,), jnp.float32)
            tacc_v[pl.ds(off, _L)] = (tacc_v[pl.ds(off, _L)]
                                      + jnp.where(valid, xt, zero))

    if nchunks > 0:
        start(0, 0, sem0)
    if nchunks > 1:
        start(1, 1, sem1)

    @pl.loop(0, nchunks, step=2)
    def pair(k):
        wait(0, sem0)
        compute(buf.at[0], c_base + k * _CHR)

        @pl.when(k + 2 < nchunks)
        def _():
            start(k + 2, 0, sem0)

        wait(1, sem1)
        compute(buf.at[1], c_base + (k + 1) * _CHR)

        @pl.when(k + 3 < nchunks)
        def _():
            start(k + 3, 1, sem1)

    pltpu.sync_copy(acc_v, acc_out.at[wid])
    pltpu.sync_copy(tacc_v, tacc_out.at[wid])


def _sc_stream(xT, tgt):
    n, b = xT.shape
    mesh = plsc.VectorSubcoreMesh(core_axis_name="c", subcore_axis_name="s",
                                  num_cores=_NC, num_subcores=_NS)
    body = functools.partial(_sc_body, b, n)
    return pl.kernel(
        body,
        out_type=(jax.ShapeDtypeStruct((_NW, b), jnp.float32),
                  jax.ShapeDtypeStruct((_NW, b), jnp.float32)),
        mesh=mesh,
        compiler_params=pltpu.CompilerParams(needs_layout_passes=False),
        scratch_types=[
            pltpu.VMEM((b,), jnp.int32),
            pltpu.VMEM((2, _CHR, b), jnp.float32),
            pltpu.VMEM((b,), jnp.float32),
            pltpu.VMEM((b,), jnp.float32),
            pltpu.SemaphoreType.DMA,
            pltpu.SemaphoreType.DMA,
        ],
    )(xT, tgt)


def _tc_body(n, t_ref, x_ref, acc_ref, tacc_ref, zacc_ref):
    j = pl.program_id(0)

    @pl.when(j == 0)
    def _init():
        acc_ref[...] = jnp.zeros_like(acc_ref)
        tacc_ref[...] = jnp.zeros_like(tacc_ref)

    x = x_ref[...]
    c0 = jnp.where(j == 0, 0, _B + (j - 1) * _RBLK)
    rid = c0 + jax.lax.broadcasted_iota(jnp.int32, x.shape, 0)
    t = t_ref[...]
    zero = jnp.zeros_like(x)
    xv = jnp.where(rid < n, x, zero)
    acc_ref[...] += jnp.sum(xv, axis=0, keepdims=True)
    tacc_ref[...] += jnp.sum(jnp.where(rid == t, x, zero), axis=0,
                             keepdims=True)

    @pl.when(j == 0)
    def _zrow():
        zacc_ref[...] = x[0:1, :]


def _tc_stream(t_row, xT):
    n, b = xT.shape
    nback = pl.cdiv(n - _B, _RBLK)
    nblocks = 1 + nback
    b_blk = _B // _RBLK
    body = functools.partial(_tc_body, n)
    return pl.pallas_call(
        body,
        grid=(nblocks,),
        in_specs=[
            pl.BlockSpec((1, b), lambda j: (0, 0)),
            pl.BlockSpec((_RBLK, b),
                         lambda j: (jnp.where(j == 0, 0, b_blk + j - 1), 0)),
        ],
        out_specs=[
            pl.BlockSpec((1, b), lambda j: (0, 0)),
            pl.BlockSpec((1, b), lambda j: (0, 0)),
            pl.BlockSpec((1, b), lambda j: (0, 0)),
        ],
        out_shape=[
            jax.ShapeDtypeStruct((1, b), jnp.float32),
            jax.ShapeDtypeStruct((1, b), jnp.float32),
            jax.ShapeDtypeStruct((1, b), jnp.float32),
        ],
    )(t_row, xT)


def _combine_body(n, t_ref, acc_tc, tacc_tc, zacc, acc_sc, tacc_sc, out_ref):
    s = _LS / (n - 2)
    k_const = _LS * math.log(s) + _CONF * math.log(_CONF)
    t = t_ref[...]
    r_total = acc_tc[...] + jnp.sum(acc_sc[...], axis=0, keepdims=True)
    xt = tacc_tc[...] + jnp.sum(tacc_sc[...], axis=0, keepdims=True)
    contrib = k_const - s * r_total + s * zacc[...] + (s - _CONF) * xt
    nonpad = t != 0
    total = jnp.sum(jnp.where(nonpad, contrib, jnp.zeros_like(contrib)))
    out_ref[...] = total.reshape(1, 1)


def kernel(output, target, extra_len):
    del extra_len
    b, n = output.shape
    xT = output.T
    tgt = target.astype(jnp.int32)
    t_row = tgt.reshape(1, b)
    acc_sc, tacc_sc = _sc_stream(xT, tgt)
    acc_tc, tacc_tc, zacc = _tc_stream(t_row, xT)
    body = functools.partial(_combine_body, n)
    res = pl.pallas_call(
        body,
        grid=(1,),
        in_specs=[
            pl.BlockSpec((1, b), lambda j: (0, 0)),
            pl.BlockSpec((1, b), lambda j: (0, 0)),
            pl.BlockSpec((1, b), lambda j: (0, 0)),
            pl.BlockSpec((1, b), lambda j: (0, 0)),
            pl.BlockSpec((_NW, b), lambda j: (0, 0)),
            pl.BlockSpec((_NW, b), lambda j: (0, 0)),
        ],
        out_specs=pl.BlockSpec((1, 1), lambda j: (0, 0)),
        out_shape=jax.ShapeDtypeStruct((1, 1), jnp.float32),
    )(t_row, acc_tc, tacc_tc, zacc, acc_sc, tacc_sc)
    return res[0, 0]


# speedup vs baseline: 1.0251x; 1.0251x over previous
import functools
import math

import jax
import jax.numpy as jnp
from jax import lax
from jax.experimental import pallas as pl
from jax.experimental.pallas import tpu as pltpu
from jax.experimental.pallas import tpu_sc as plsc

_LS = 0.1
_CONF = 1.0 - _LS
_RBLK = 2048
_A = 2048
_B = 45056
_CHR = 32

_NC = 2
_NS = 16
_L = 16
_NW = _NC * _NS


def _sc_body(b, n, xT, tgt, acc_out, tacc_out, tgt_v, buf, acc_v, tacc_v,
             sem0, sem1):
    w_per = (_B - _A) // _NW
    nchunks = w_per // _CHR
    ngroups = b // _L
    wid = lax.axis_index("s") * _NC + lax.axis_index("c")
    c_base = _A + wid * w_per

    pltpu.sync_copy(tgt, tgt_v)

    @pl.loop(0, ngroups)
    def _zero(j):
        off = j * _L
        acc_v[pl.ds(off, _L)] = jnp.zeros((_L,), jnp.float32)
        tacc_v[pl.ds(off, _L)] = jnp.zeros((_L,), jnp.float32)

    def start(k, bsel, sem):
        pltpu.make_async_copy(
            xT.at[pl.ds(c_base + k * _CHR, _CHR), :],
            buf.at[bsel], sem).start()

    def wait(bsel, sem):
        pltpu.make_async_copy(
            xT.at[pl.ds(0, _CHR), :], buf.at[bsel], sem).wait()

    def compute(bufref, c0):
        @pl.loop(0, ngroups)
        def _grp(j):
            off = j * _L
            a0 = bufref[0, pl.ds(off, _L)]
            a1 = bufref[1, pl.ds(off, _L)]
            a2 = bufref[2, pl.ds(off, _L)]
            a3 = bufref[3, pl.ds(off, _L)]
            for r in range(4, _CHR, 4):
                a0 = a0 + bufref[r, pl.ds(off, _L)]
                a1 = a1 + bufref[r + 1, pl.ds(off, _L)]
                a2 = a2 + bufref[r + 2, pl.ds(off, _L)]
                a3 = a3 + bufref[r + 3, pl.ds(off, _L)]
            acc_v[pl.ds(off, _L)] = (acc_v[pl.ds(off, _L)]
                                     + (a0 + a1) + (a2 + a3))

            t16 = tgt_v[pl.ds(off, _L)]
            rel = t16 - c0
            valid = (rel >= 0) & (rel < _CHR)
            relc = jnp.minimum(jnp.maximum(rel, 0), _CHR - 1)
            bcol = off + lax.iota(jnp.int32, _L)
            xt = plsc.load_gather(bufref, [relc, bcol])
            zero = jnp.zeros((_L,), jnp.float32)
            tacc_v[pl.ds(off, _L)] = (tacc_v[pl.ds(off, _L)]
                                      + jnp.where(valid, xt, zero))

    if nchunks > 0:
        start(0, 0, sem0)
    if nchunks > 1:
        start(1, 1, sem1)

    @pl.loop(0, nchunks, step=2)
    def pair(k):
        wait(0, sem0)
        compute(buf.at[0], c_base + k * _CHR)

        @pl.when(k + 2 < nchunks)
        def _():
            start(k + 2, 0, sem0)

        wait(1, sem1)
        compute(buf.at[1], c_base + (k + 1) * _CHR)

        @pl.when(k + 3 < nchunks)
        def _():
            start(k + 3, 1, sem1)

    pltpu.sync_copy(acc_v, acc_out.at[wid])
    pltpu.sync_copy(tacc_v, tacc_out.at[wid])


def _sc_stream(xT, tgt):
    n, b = xT.shape
    mesh = plsc.VectorSubcoreMesh(core_axis_name="c", subcore_axis_name="s",
                                  num_cores=_NC, num_subcores=_NS)
    body = functools.partial(_sc_body, b, n)
    return pl.kernel(
        body,
        out_type=(jax.ShapeDtypeStruct((_NW, b), jnp.float32),
                  jax.ShapeDtypeStruct((_NW, b), jnp.float32)),
        mesh=mesh,
        compiler_params=pltpu.CompilerParams(needs_layout_passes=False),
        scratch_types=[
            pltpu.VMEM((b,), jnp.int32),
            pltpu.VMEM((2, _CHR, b), jnp.float32),
            pltpu.VMEM((b,), jnp.float32),
            pltpu.VMEM((b,), jnp.float32),
            pltpu.SemaphoreType.DMA,
            pltpu.SemaphoreType.DMA,
        ],
    )(xT, tgt)


def _tc_body(n, t_ref, x_ref, acc_ref, tacc_ref, zacc_ref):
    j = pl.program_id(0)

    @pl.when(j == 0)
    def _init():
        acc_ref[...] = jnp.zeros_like(acc_ref)
        tacc_ref[...] = jnp.zeros_like(tacc_ref)

    x = x_ref[...]
    c0 = jnp.where(j == 0, 0, _B + (j - 1) * _RBLK)
    rid = c0 + jax.lax.broadcasted_iota(jnp.int32, x.shape, 0)
    t = t_ref[...]
    zero = jnp.zeros_like(x)
    xv = jnp.where(rid < n, x, zero)
    acc_ref[...] += jnp.sum(xv, axis=0, keepdims=True)
    tacc_ref[...] += jnp.sum(jnp.where(rid == t, x, zero), axis=0,
                             keepdims=True)

    @pl.when(j == 0)
    def _zrow():
        zacc_ref[...] = x[0:1, :]


def _tc_stream(t_row, xT):
    n, b = xT.shape
    nback = pl.cdiv(n - _B, _RBLK)
    nblocks = 1 + nback
    b_blk = _B // _RBLK
    body = functools.partial(_tc_body, n)
    return pl.pallas_call(
        body,
        grid=(nblocks,),
        in_specs=[
            pl.BlockSpec((1, b), lambda j: (0, 0)),
            pl.BlockSpec((_RBLK, b),
                         lambda j: (jnp.where(j == 0, 0, b_blk + j - 1), 0)),
        ],
        out_specs=[
            pl.BlockSpec((1, b), lambda j: (0, 0)),
            pl.BlockSpec((1, b), lambda j: (0, 0)),
            pl.BlockSpec((1, b), lambda j: (0, 0)),
        ],
        out_shape=[
            jax.ShapeDtypeStruct((1, b), jnp.float32),
            jax.ShapeDtypeStruct((1, b), jnp.float32),
            jax.ShapeDtypeStruct((1, b), jnp.float32),
        ],
    )(t_row, xT)


def _combine_body(n, t_ref, acc_tc, tacc_tc, zacc, acc_sc, tacc_sc, out_ref):
    s = _LS / (n - 2)
    k_const = _LS * math.log(s) + _CONF * math.log(_CONF)
    t = t_ref[...]
    r_total = acc_tc[...] + jnp.sum(acc_sc[...], axis=0, keepdims=True)
    xt = tacc_tc[...] + jnp.sum(tacc_sc[...], axis=0, keepdims=True)
    contrib = k_const - s * r_total + s * zacc[...] + (s - _CONF) * xt
    nonpad = t != 0
    total = jnp.sum(jnp.where(nonpad, contrib, jnp.zeros_like(contrib)))
    out_ref[...] = total.reshape(1, 1)


def kernel(output, target, extra_len):
    del extra_len
    b, n = output.shape
    xT = output.T
    tgt = target.astype(jnp.int32)
    t_row = tgt.reshape(1, b)
    acc_sc, tacc_sc = _sc_stream(xT, tgt)
    acc_tc, tacc_tc, zacc = _tc_stream(t_row, xT)
    body = functools.partial(_combine_body, n)
    res = pl.pallas_call(
        body,
        grid=(1,),
        in_specs=[
            pl.BlockSpec((1, b), lambda j: (0, 0)),
            pl.BlockSpec((1, b), lambda j: (0, 0)),
            pl.BlockSpec((1, b), lambda j: (0, 0)),
            pl.BlockSpec((1, b), lambda j: (0, 0)),
            pl.BlockSpec((_NW, b), lambda j: (0, 0)),
            pl.BlockSpec((_NW, b), lambda j: (0, 0)),
        ],
        out_specs=pl.BlockSpec((1, 1), lambda j: (0, 0)),
        out_shape=jax.ShapeDtypeStruct((1, 1), jnp.float32),
    )(t_row, acc_tc, tacc_tc, zacc, acc_sc, tacc_sc)
    return res[0, 0]


# speedup vs baseline: 1.0340x; 1.0087x over previous
import functools
import math

import jax
import jax.numpy as jnp
from jax import lax
from jax.experimental import pallas as pl
from jax.experimental.pallas import tpu as pltpu
from jax.experimental.pallas import tpu_sc as plsc

_LS = 0.1
_CONF = 1.0 - _LS
_RBLK = 2048
_A = 2048
_B = 43008
_CHR = 32

_NC = 2
_NS = 16
_L = 16
_NW = _NC * _NS


def _sc_body(b, n, xT, tgt, acc_out, tacc_out, tgt_v, buf, acc_v, tacc_v,
             sem0, sem1):
    w_per = (_B - _A) // _NW
    nchunks = w_per // _CHR
    ngroups = b // _L
    wid = lax.axis_index("s") * _NC + lax.axis_index("c")
    c_base = _A + wid * w_per

    pltpu.sync_copy(tgt, tgt_v)

    @pl.loop(0, ngroups)
    def _zero(j):
        off = j * _L
        acc_v[pl.ds(off, _L)] = jnp.zeros((_L,), jnp.float32)
        tacc_v[pl.ds(off, _L)] = jnp.zeros((_L,), jnp.float32)

    def start(k, bsel, sem):
        pltpu.make_async_copy(
            xT.at[pl.ds(c_base + k * _CHR, _CHR), :],
            buf.at[bsel], sem).start()

    def wait(bsel, sem):
        pltpu.make_async_copy(
            xT.at[pl.ds(0, _CHR), :], buf.at[bsel], sem).wait()

    def compute(bufref, c0):
        @pl.loop(0, ngroups)
        def _grp(j):
            off = j * _L
            a0 = bufref[0, pl.ds(off, _L)]
            a1 = bufref[1, pl.ds(off, _L)]
            a2 = bufref[2, pl.ds(off, _L)]
            a3 = bufref[3, pl.ds(off, _L)]
            for r in range(4, _CHR, 4):
                a0 = a0 + bufref[r, pl.ds(off, _L)]
                a1 = a1 + bufref[r + 1, pl.ds(off, _L)]
                a2 = a2 + bufref[r + 2, pl.ds(off, _L)]
                a3 = a3 + bufref[r + 3, pl.ds(off, _L)]
            acc_v[pl.ds(off, _L)] = (acc_v[pl.ds(off, _L)]
                                     + (a0 + a1) + (a2 + a3))

            t16 = tgt_v[pl.ds(off, _L)]
            rel = t16 - c0
            valid = (rel >= 0) & (rel < _CHR)
            relc = jnp.minimum(jnp.maximum(rel, 0), _CHR - 1)
            bcol = off + lax.iota(jnp.int32, _L)
            xt = plsc.load_gather(bufref, [relc, bcol])
            zero = jnp.zeros((_L,), jnp.float32)
            tacc_v[pl.ds(off, _L)] = (tacc_v[pl.ds(off, _L)]
                                      + jnp.where(valid, xt, zero))

    if nchunks > 0:
        start(0, 0, sem0)
    if nchunks > 1:
        start(1, 1, sem1)

    @pl.loop(0, nchunks, step=2)
    def pair(k):
        wait(0, sem0)
        compute(buf.at[0], c_base + k * _CHR)

        @pl.when(k + 2 < nchunks)
        def _():
            start(k + 2, 0, sem0)

        wait(1, sem1)
        compute(buf.at[1], c_base + (k + 1) * _CHR)

        @pl.when(k + 3 < nchunks)
        def _():
            start(k + 3, 1, sem1)

    pltpu.sync_copy(acc_v, acc_out.at[wid])
    pltpu.sync_copy(tacc_v, tacc_out.at[wid])


def _sc_stream(xT, tgt):
    n, b = xT.shape
    mesh = plsc.VectorSubcoreMesh(core_axis_name="c", subcore_axis_name="s",
                                  num_cores=_NC, num_subcores=_NS)
    body = functools.partial(_sc_body, b, n)
    return pl.kernel(
        body,
        out_type=(jax.ShapeDtypeStruct((_NW, b), jnp.float32),
                  jax.ShapeDtypeStruct((_NW, b), jnp.float32)),
        mesh=mesh,
        compiler_params=pltpu.CompilerParams(needs_layout_passes=False),
        scratch_types=[
            pltpu.VMEM((b,), jnp.int32),
            pltpu.VMEM((2, _CHR, b), jnp.float32),
            pltpu.VMEM((b,), jnp.float32),
            pltpu.VMEM((b,), jnp.float32),
            pltpu.SemaphoreType.DMA,
            pltpu.SemaphoreType.DMA,
        ],
    )(xT, tgt)


def _tc_body(n, t_ref, x_ref, acc_ref, tacc_ref, zacc_ref):
    j = pl.program_id(0)

    @pl.when(j == 0)
    def _init():
        acc_ref[...] = jnp.zeros_like(acc_ref)
        tacc_ref[...] = jnp.zeros_like(tacc_ref)

    x = x_ref[...]
    c0 = jnp.where(j == 0, 0, _B + (j - 1) * _RBLK)
    rid = c0 + jax.lax.broadcasted_iota(jnp.int32, x.shape, 0)
    t = t_ref[...]
    zero = jnp.zeros_like(x)
    xv = jnp.where(rid < n, x, zero)
    acc_ref[...] += jnp.sum(xv, axis=0, keepdims=True)
    tacc_ref[...] += jnp.sum(jnp.where(rid == t, x, zero), axis=0,
                             keepdims=True)

    @pl.when(j == 0)
    def _zrow():
        zacc_ref[...] = x[0:1, :]


def _tc_stream(t_row, xT):
    n, b = xT.shape
    nback = pl.cdiv(n - _B, _RBLK)
    nblocks = 1 + nback
    b_blk = _B // _RBLK
    body = functools.partial(_tc_body, n)
    return pl.pallas_call(
        body,
        grid=(nblocks,),
        in_specs=[
            pl.BlockSpec((1, b), lambda j: (0, 0)),
            pl.BlockSpec((_RBLK, b),
                         lambda j: (jnp.where(j == 0, 0, b_blk + j - 1), 0)),
        ],
        out_specs=[
            pl.BlockSpec((1, b), lambda j: (0, 0)),
            pl.BlockSpec((1, b), lambda j: (0, 0)),
            pl.BlockSpec((1, b), lambda j: (0, 0)),
        ],
        out_shape=[
            jax.ShapeDtypeStruct((1, b), jnp.float32),
            jax.ShapeDtypeStruct((1, b), jnp.float32),
            jax.ShapeDtypeStruct((1, b), jnp.float32),
        ],
    )(t_row, xT)


def _combine_body(n, t_ref, acc_tc, tacc_tc, zacc, acc_sc, tacc_sc, out_ref):
    s = _LS / (n - 2)
    k_const = _LS * math.log(s) + _CONF * math.log(_CONF)
    t = t_ref[...]
    r_total = acc_tc[...] + jnp.sum(acc_sc[...], axis=0, keepdims=True)
    xt = tacc_tc[...] + jnp.sum(tacc_sc[...], axis=0, keepdims=True)
    contrib = k_const - s * r_total + s * zacc[...] + (s - _CONF) * xt
    nonpad = t != 0
    total = jnp.sum(jnp.where(nonpad, contrib, jnp.zeros_like(contrib)))
    out_ref[...] = total.reshape(1, 1)


def kernel(output, target, extra_len):
    del extra_len
    b, n = output.shape
    xT = output.T
    tgt = target.astype(jnp.int32)
    t_row = tgt.reshape(1, b)
    acc_sc, tacc_sc = _sc_stream(xT, tgt)
    acc_tc, tacc_tc, zacc = _tc_stream(t_row, xT)
    body = functools.partial(_combine_body, n)
    res = pl.pallas_call(
        body,
        grid=(1,),
        in_specs=[
            pl.BlockSpec((1, b), lambda j: (0, 0)),
            pl.BlockSpec((1, b), lambda j: (0, 0)),
            pl.BlockSpec((1, b), lambda j: (0, 0)),
            pl.BlockSpec((1, b), lambda j: (0, 0)),
            pl.BlockSpec((_NW, b), lambda j: (0, 0)),
            pl.BlockSpec((_NW, b), lambda j: (0, 0)),
        ],
        out_specs=pl.BlockSpec((1, 1), lambda j: (0, 0)),
        out_shape=jax.ShapeDtypeStruct((1, 1), jnp.float32),
    )(t_row, acc_tc, tacc_tc, zacc, acc_sc, tacc_sc)
    return res[0, 0]
